# baseline (speedup 1.0000x reference)
import jax
import jax.numpy as jnp
from jax.experimental import pallas as pl
from jax.experimental.pallas import tpu as pltpu

_GAMMA1 = 1.0


def _kf_kernel(x_ref, d_ref, xrec_ref, loss_ref, label_ref, bd2_ref):
    n = pl.program_id(0)
    num_n = pl.num_programs(0)
    x = x_ref[...]
    dn = d_ref[0]
    cs = jnp.dot(x, dn, preferred_element_type=jnp.float32)
    xh = jnp.dot(cs, dn.T, preferred_element_type=jnp.float32)
    diff = xh - x
    d2 = jnp.sum(diff * diff, axis=1, keepdims=True)

    @pl.when(n == 0)
    def _():
        bd2_ref[...] = d2
        xrec_ref[...] = xh
        label_ref[...] = jnp.zeros_like(label_ref)

    @pl.when(n > 0)
    def _():
        better = d2 < bd2_ref[...]
        bd2_ref[...] = jnp.where(better, d2, bd2_ref[...])
        xrec_ref[...] = jnp.where(better, xh, xrec_ref[...])
        label_ref[...] = jnp.where(better, n, label_ref[...])

    @pl.when(n == num_n - 1)
    def _():
        r = xrec_ref[...] - x
        loss_ref[...] = (jnp.mean(r * r) * _GAMMA1).reshape(1, 1)


def kernel(x, D):
    B, dim = x.shape
    N, _, F = D.shape
    x_rec, loss, label = pl.pallas_call(
        _kf_kernel,
        grid=(N,),
        in_specs=[
            pl.BlockSpec((B, dim), lambda n: (0, 0)),
            pl.BlockSpec((1, dim, F), lambda n: (n, 0, 0)),
        ],
        out_specs=[
            pl.BlockSpec((B, dim), lambda n: (0, 0)),
            pl.BlockSpec((1, 1), lambda n: (0, 0)),
            pl.BlockSpec((B, 1), lambda n: (0, 0)),
        ],
        out_shape=[
            jax.ShapeDtypeStruct((B, dim), jnp.float32),
            jax.ShapeDtypeStruct((1, 1), jnp.float32),
            jax.ShapeDtypeStruct((B, 1), jnp.int32),
        ],
        scratch_shapes=[pltpu.VMEM((B, 1), jnp.float32)],
    )(x, D)
    return (x_rec, loss[0, 0], label[:, 0])


# speedup vs baseline: 1.0082x; 1.0082x over previous
import jax
import jax.numpy as jnp
from jax.experimental import pallas as pl
from jax.experimental.pallas import tpu as pltpu

_GAMMA1 = 1.0


def _kf_kernel(x_ref, d_ref, xrec_ref, loss_ref, label_ref, bd2_ref):
    n = pl.program_id(0)
    num_n = pl.num_programs(0)
    x = x_ref[...]
    dn = d_ref[0]
    cs = jnp.dot(x, dn, preferred_element_type=jnp.float32,
                 precision=jax.lax.Precision.DEFAULT)
    xh = jnp.dot(cs, dn.T, preferred_element_type=jnp.float32,
                 precision=jax.lax.Precision.DEFAULT)
    diff = xh - x
    d2 = jnp.sum(diff * diff, axis=1, keepdims=True)

    @pl.when(n == 0)
    def _():
        bd2_ref[...] = d2
        xrec_ref[...] = xh
        label_ref[...] = jnp.zeros_like(label_ref)

    @pl.when(n > 0)
    def _():
        better = d2 < bd2_ref[...]
        bd2_ref[...] = jnp.where(better, d2, bd2_ref[...])
        xrec_ref[...] = jnp.where(better, xh, xrec_ref[...])
        label_ref[...] = jnp.where(better, n, label_ref[...])

    @pl.when(n == num_n - 1)
    def _():
        r = xrec_ref[...] - x
        loss_ref[...] = (jnp.mean(r * r) * _GAMMA1).reshape(1, 1)


def kernel(x, D):
    B, dim = x.shape
    N, _, F = D.shape
    x_rec, loss, label = pl.pallas_call(
        _kf_kernel,
        grid=(N,),
        in_specs=[
            pl.BlockSpec((B, dim), lambda n: (0, 0)),
            pl.BlockSpec((1, dim, F), lambda n: (n, 0, 0)),
        ],
        out_specs=[
            pl.BlockSpec((B, dim), lambda n: (0, 0)),
            pl.BlockSpec((1, 1), lambda n: (0, 0)),
            pl.BlockSpec((B, 1), lambda n: (0, 0)),
        ],
        out_shape=[
            jax.ShapeDtypeStruct((B, dim), jnp.float32),
            jax.ShapeDtypeStruct((1, 1), jnp.float32),
            jax.ShapeDtypeStruct((B, 1), jnp.int32),
        ],
        scratch_shapes=[pltpu.VMEM((B, 1), jnp.float32)],
    )(x, D)
    return (x_rec, loss[0, 0], label[:, 0])


# speedup vs baseline: 1.5807x; 1.5678x over previous
import jax
import jax.numpy as jnp
from jax.experimental import pallas as pl
from jax.experimental.pallas import tpu as pltpu

_GAMMA1 = 1.0


_NB = 2


def _kf_kernel(x_ref, d_ref, xrec_ref, loss_ref, label_ref, bd2_ref):
    n = pl.program_id(0)
    num_n = pl.num_programs(0)
    x = x_ref[...]
    d2s, xhs = [], []
    for j in range(_NB):
        dn = d_ref[j]
        cs = jnp.dot(x, dn, preferred_element_type=jnp.float32,
                     precision=jax.lax.Precision.DEFAULT)
        xh = jnp.dot(cs, dn.T, preferred_element_type=jnp.float32,
                     precision=jax.lax.Precision.DEFAULT)
        diff = xh - x
        d2s.append(jnp.sum(diff * diff, axis=1, keepdims=True))
        xhs.append(xh)

    d2, xh = d2s[0], xhs[0]
    lbl = jnp.zeros_like(d2, dtype=jnp.int32) + _NB * n
    for j in range(1, _NB):
        better = d2s[j] < d2
        d2 = jnp.where(better, d2s[j], d2)
        xh = jnp.where(better, xhs[j], xh)
        lbl = jnp.where(better, _NB * n + j, lbl)

    @pl.when(n == 0)
    def _():
        bd2_ref[...] = d2
        xrec_ref[...] = xh
        label_ref[...] = lbl

    @pl.when(n > 0)
    def _():
        better = d2 < bd2_ref[...]
        bd2_ref[...] = jnp.where(better, d2, bd2_ref[...])
        xrec_ref[...] = jnp.where(better, xh, xrec_ref[...])
        label_ref[...] = jnp.where(better, lbl, label_ref[...])

    @pl.when(n == num_n - 1)
    def _():
        r = xrec_ref[...] - x
        loss_ref[...] = (jnp.mean(r * r) * _GAMMA1).reshape(1, 1)


def kernel(x, D):
    B, dim = x.shape
    N, _, F = D.shape
    x_rec, loss, label = pl.pallas_call(
        _kf_kernel,
        grid=(N // _NB,),
        in_specs=[
            pl.BlockSpec((B, dim), lambda n: (0, 0)),
            pl.BlockSpec((_NB, dim, F), lambda n: (n, 0, 0)),
        ],
        out_specs=[
            pl.BlockSpec((B, dim), lambda n: (0, 0)),
            pl.BlockSpec((1, 1), lambda n: (0, 0)),
            pl.BlockSpec((B, 1), lambda n: (0, 0)),
        ],
        out_shape=[
            jax.ShapeDtypeStruct((B, dim), jnp.float32),
            jax.ShapeDtypeStruct((1, 1), jnp.float32),
            jax.ShapeDtypeStruct((B, 1), jnp.int32),
        ],
        scratch_shapes=[pltpu.VMEM((B, 1), jnp.float32)],
    )(x, D)
    return (x_rec, loss[0, 0], label[:, 0])


# speedup vs baseline: 2.0967x; 1.3265x over previous
import jax
import jax.numpy as jnp
from jax.experimental import pallas as pl
from jax.experimental.pallas import tpu as pltpu

_GAMMA1 = 1.0


_NB = 4


def _kf_kernel(x_ref, d_ref, xrec_ref, loss_ref, label_ref, bd2_ref):
    n = pl.program_id(0)
    num_n = pl.num_programs(0)
    x = x_ref[...]
    d2s, xhs = [], []
    for j in range(_NB):
        dn = d_ref[j]
        cs = jnp.dot(x, dn, preferred_element_type=jnp.float32,
                     precision=jax.lax.Precision.DEFAULT)
        xh = jnp.dot(cs, dn.T, preferred_element_type=jnp.float32,
                     precision=jax.lax.Precision.DEFAULT)
        diff = xh - x
        d2s.append(jnp.sum(diff * diff, axis=1, keepdims=True))
        xhs.append(xh)

    d2, xh = d2s[0], xhs[0]
    lbl = jnp.zeros_like(d2, dtype=jnp.int32) + _NB * n
    for j in range(1, _NB):
        better = d2s[j] < d2
        d2 = jnp.where(better, d2s[j], d2)
        xh = jnp.where(better, xhs[j], xh)
        lbl = jnp.where(better, _NB * n + j, lbl)

    @pl.when(n == 0)
    def _():
        bd2_ref[...] = d2
        xrec_ref[...] = xh
        label_ref[...] = lbl

    @pl.when(n > 0)
    def _():
        better = d2 < bd2_ref[...]
        bd2_ref[...] = jnp.where(better, d2, bd2_ref[...])
        xrec_ref[...] = jnp.where(better, xh, xrec_ref[...])
        label_ref[...] = jnp.where(better, lbl, label_ref[...])

    @pl.when(n == num_n - 1)
    def _():
        r = xrec_ref[...] - x
        loss_ref[...] = (jnp.mean(r * r) * _GAMMA1).reshape(1, 1)


def kernel(x, D):
    B, dim = x.shape
    N, _, F = D.shape
    x_rec, loss, label = pl.pallas_call(
        _kf_kernel,
        grid=(N // _NB,),
        in_specs=[
            pl.BlockSpec((B, dim), lambda n: (0, 0)),
            pl.BlockSpec((_NB, dim, F), lambda n: (n, 0, 0)),
        ],
        out_specs=[
            pl.BlockSpec((B, dim), lambda n: (0, 0)),
            pl.BlockSpec((1, 1), lambda n: (0, 0)),
            pl.BlockSpec((B, 1), lambda n: (0, 0)),
        ],
        out_shape=[
            jax.ShapeDtypeStruct((B, dim), jnp.float32),
            jax.ShapeDtypeStruct((1, 1), jnp.float32),
            jax.ShapeDtypeStruct((B, 1), jnp.int32),
        ],
        scratch_shapes=[pltpu.VMEM((B, 1), jnp.float32)],
    )(x, D)
    return (x_rec, loss[0, 0], label[:, 0])


# speedup vs baseline: 2.4648x; 1.1755x over previous
import jax
import jax.numpy as jnp
from jax.experimental import pallas as pl
from jax.experimental.pallas import tpu as pltpu

_GAMMA1 = 1.0


_NB = 8


def _kf_kernel(x_ref, d_ref, xrec_ref, loss_ref, label_ref, bd2_ref):
    n = pl.program_id(0)
    num_n = pl.num_programs(0)
    x = x_ref[...]
    d2s, xhs = [], []
    for j in range(_NB):
        dn = d_ref[j]
        cs = jnp.dot(x, dn, preferred_element_type=jnp.float32,
                     precision=jax.lax.Precision.DEFAULT)
        xh = jnp.dot(cs, dn.T, preferred_element_type=jnp.float32,
                     precision=jax.lax.Precision.DEFAULT)
        diff = xh - x
        d2s.append(jnp.sum(diff * diff, axis=1, keepdims=True))
        xhs.append(xh)

    d2, xh = d2s[0], xhs[0]
    lbl = jnp.zeros_like(d2, dtype=jnp.int32) + _NB * n
    for j in range(1, _NB):
        better = d2s[j] < d2
        d2 = jnp.where(better, d2s[j], d2)
        xh = jnp.where(better, xhs[j], xh)
        lbl = jnp.where(better, _NB * n + j, lbl)

    @pl.when(n == 0)
    def _():
        bd2_ref[...] = d2
        xrec_ref[...] = xh
        label_ref[...] = lbl

    @pl.when(n > 0)
    def _():
        better = d2 < bd2_ref[...]
        bd2_ref[...] = jnp.where(better, d2, bd2_ref[...])
        xrec_ref[...] = jnp.where(better, xh, xrec_ref[...])
        label_ref[...] = jnp.where(better, lbl, label_ref[...])

    @pl.when(n == num_n - 1)
    def _():
        r = xrec_ref[...] - x
        loss_ref[...] = (jnp.mean(r * r) * _GAMMA1).reshape(1, 1)


def kernel(x, D):
    B, dim = x.shape
    N, _, F = D.shape
    x_rec, loss, label = pl.pallas_call(
        _kf_kernel,
        grid=(N // _NB,),
        in_specs=[
            pl.BlockSpec((B, dim), lambda n: (0, 0)),
            pl.BlockSpec((_NB, dim, F), lambda n: (n, 0, 0)),
        ],
        out_specs=[
            pl.BlockSpec((B, dim), lambda n: (0, 0)),
            pl.BlockSpec((1, 1), lambda n: (0, 0)),
            pl.BlockSpec((B, 1), lambda n: (0, 0)),
        ],
        out_shape=[
            jax.ShapeDtypeStruct((B, dim), jnp.float32),
            jax.ShapeDtypeStruct((1, 1), jnp.float32),
            jax.ShapeDtypeStruct((B, 1), jnp.int32),
        ],
        scratch_shapes=[pltpu.VMEM((B, 1), jnp.float32)],
    )(x, D)
    return (x_rec, loss[0, 0], label[:, 0])


# speedup vs baseline: 2.5906x; 1.0510x over previous
import jax
import jax.numpy as jnp
from jax.experimental import pallas as pl
from jax.experimental.pallas import tpu as pltpu

_GAMMA1 = 1.0


_NB = 16


def _kf_kernel(x_ref, d_ref, xrec_ref, loss_ref, label_ref, bd2_ref):
    n = pl.program_id(0)
    num_n = pl.num_programs(0)
    x = x_ref[...]
    d2s, xhs = [], []
    for j in range(_NB):
        dn = d_ref[j]
        cs = jnp.dot(x, dn, preferred_element_type=jnp.float32,
                     precision=jax.lax.Precision.DEFAULT)
        xh = jnp.dot(cs, dn.T, preferred_element_type=jnp.float32,
                     precision=jax.lax.Precision.DEFAULT)
        diff = xh - x
        d2s.append(jnp.sum(diff * diff, axis=1, keepdims=True))
        xhs.append(xh)

    d2, xh = d2s[0], xhs[0]
    lbl = jnp.zeros_like(d2, dtype=jnp.int32) + _NB * n
    for j in range(1, _NB):
        better = d2s[j] < d2
        d2 = jnp.where(better, d2s[j], d2)
        xh = jnp.where(better, xhs[j], xh)
        lbl = jnp.where(better, _NB * n + j, lbl)

    @pl.when(n == 0)
    def _():
        bd2_ref[...] = d2
        xrec_ref[...] = xh
        label_ref[...] = lbl

    @pl.when(n > 0)
    def _():
        better = d2 < bd2_ref[...]
        bd2_ref[...] = jnp.where(better, d2, bd2_ref[...])
        xrec_ref[...] = jnp.where(better, xh, xrec_ref[...])
        label_ref[...] = jnp.where(better, lbl, label_ref[...])

    @pl.when(n == num_n - 1)
    def _():
        r = xrec_ref[...] - x
        loss_ref[...] = (jnp.mean(r * r) * _GAMMA1).reshape(1, 1)


def kernel(x, D):
    B, dim = x.shape
    N, _, F = D.shape
    x_rec, loss, label = pl.pallas_call(
        _kf_kernel,
        grid=(N // _NB,),
        in_specs=[
            pl.BlockSpec((B, dim), lambda n: (0, 0)),
            pl.BlockSpec((_NB, dim, F), lambda n: (n, 0, 0)),
        ],
        out_specs=[
            pl.BlockSpec((B, dim), lambda n: (0, 0)),
            pl.BlockSpec((1, 1), lambda n: (0, 0)),
            pl.BlockSpec((B, 1), lambda n: (0, 0)),
        ],
        out_shape=[
            jax.ShapeDtypeStruct((B, dim), jnp.float32),
            jax.ShapeDtypeStruct((1, 1), jnp.float32),
            jax.ShapeDtypeStruct((B, 1), jnp.int32),
        ],
        scratch_shapes=[pltpu.VMEM((B, 1), jnp.float32)],
    )(x, D)
    return (x_rec, loss[0, 0], label[:, 0])
